# baseline (speedup 1.0000x reference)
import jax
import jax.numpy as jnp
from jax.experimental import pallas as pl
from jax.experimental.pallas import tpu as pltpu

_B, _S, _D, _M = 8, 8, 128, 4096
_TOPK = 8
_CAP = _S * _TOPK


def _layer_body(xs_ref, mem_ref, wqT_ref, wkT_ref, wvT_ref, wuxT_ref, wumT_ref,
                bq_ref, bv_ref, bu_ref, out_ref,
                cur_ref, lg_ref, tidx_ref, tdel_ref):
    f32 = jnp.float32
    inv_scale = f32(1.0 / (_D ** 0.5))
    neg_inf = f32(-jnp.inf)

    for b in range(_B):
        cur_ref[b] = mem_ref[...]
        tdel_ref[b] = jnp.zeros((_CAP, _D), f32)
    tidx_ref[...] = jnp.zeros((_CAP, _B), jnp.int32)

    iota = jax.lax.broadcasted_iota(jnp.int32, (_B, _M), 1)
    iota_cap = jax.lax.broadcasted_iota(jnp.int32, (_CAP, 1), 0)

    def step(t, carry):
        xt = xs_ref[pl.ds(t, 1), :, :].reshape(_B, _D)
        q = jnp.dot(xt, wqT_ref[...], preferred_element_type=f32) + bq_ref[...]
        wT = jnp.dot(wkT_ref[...], q.T, preferred_element_type=f32)

        lg_ref[...] = jnp.dot(mem_ref[...], wT, preferred_element_type=f32)
        for b in range(_B):
            cv = jnp.dot(tdel_ref[b], wT[:, b:b + 1],
                         preferred_element_type=f32)
            for j in range(_CAP):
                i = tidx_ref[j, b]
                lg_ref[pl.ds(i, 1), b:b + 1] = (
                    lg_ref[pl.ds(i, 1), b:b + 1] + cv[j, 0])

        logits = lg_ref[...].T * inv_scale

        mx = jnp.max(logits, axis=1, keepdims=True)
        e = jnp.exp(logits - mx)
        attn = e * (1.0 / jnp.sum(e, axis=1, keepdims=True))

        rows = [jnp.dot(attn[b:b + 1, :], cur_ref[b], preferred_element_type=f32)
                for b in range(_B)]
        s = jnp.concatenate(rows, axis=0)
        mem_out = jnp.dot(s, wvT_ref[...], preferred_element_type=f32) + bv_ref[...]
        out_ref[pl.ds(t, 1), :, :] = mem_out.reshape(1, _B, _D)

        u = jax.nn.sigmoid(jnp.dot(xt, wuxT_ref[...], preferred_element_type=f32)
                           + jnp.dot(mem_out, wumT_ref[...], preferred_element_type=f32)
                           + bu_ref[...])

        lg = logits
        top = []
        for _ in range(_TOPK):
            mxj = jnp.max(lg, axis=1, keepdims=True)
            cand = jnp.where(lg >= mxj, iota, _M)
            ij = jnp.min(cand, axis=1, keepdims=True)
            top.append(ij)
            lg = jnp.where(iota == ij, neg_inf, lg)

        zero_row = jnp.zeros((1, _D), f32)
        for b in range(_B):
            ub = u[b:b + 1, :]
            xb = xt[b:b + 1, :]
            tcol = tidx_ref[:, b:b + 1]
            for j in range(_TOPK):
                i = top[j][b, 0]
                g = cur_ref[b, pl.ds(i, 1), :]
                new = g + ub * (xb - g)
                cur_ref[b, pl.ds(i, 1), :] = new
                slot = t * _TOPK + j
                pos = jnp.max(jnp.where(tcol == i, iota_cap, -1))
                tgt = jnp.where(pos >= 0, pos, slot)
                tdel_ref[b, pl.ds(tgt, 1), :] = zero_row
                tdel_ref[b, pl.ds(slot, 1), :] = new - mem_ref[pl.ds(i, 1), :]
                tidx_ref[pl.ds(slot, 1), b:b + 1] = jnp.full((1, 1), i, jnp.int32)
        return carry

    jax.lax.fori_loop(0, _S, step, 0)


def _run(xs, memory, wqT, wkT, wvT, wuxT, wumT, bq2, bv2, bu2):
    return pl.pallas_call(
        _layer_body,
        out_shape=jax.ShapeDtypeStruct((_S, _B, _D), jnp.float32),
        scratch_shapes=[
            pltpu.VMEM((_B, _M, _D), jnp.float32),
            pltpu.VMEM((_M, _B), jnp.float32),
            pltpu.VMEM((_CAP, _B), jnp.int32),
            pltpu.VMEM((_B, _CAP, _D), jnp.float32),
        ],
    )(xs, memory, wqT, wkT, wvT, wuxT, wumT, bq2, bv2, bu2)


def kernel(x, memory, Wq, bq, Wk, bk, Wv, bv, Wu, bu):
    del bk
    xs = jnp.transpose(x, (1, 0, 2))
    wqT = Wq.T
    wkT = Wk.T
    wvT = Wv.T
    wuxT = Wu[:, :_D].T
    wumT = Wu[:, _D:].T
    bq2 = bq.reshape(1, _D)
    bv2 = bv.reshape(1, _D)
    bu2 = bu.reshape(1, _D)
    outs = _run(xs, memory, wqT, wkT, wvT, wuxT, wumT, bq2, bv2, bu2)
    return jnp.transpose(outs, (1, 0, 2))


# speedup vs baseline: 2.9814x; 2.9814x over previous
import jax
import jax.numpy as jnp
from jax.experimental import pallas as pl
from jax.experimental.pallas import tpu as pltpu

_B, _S, _D, _M = 8, 8, 128, 4096
_TOPK = 8


def _layer_body(xs_ref, mem_ref, wqT_ref, wk_ref, wvT_ref, wuxT_ref, wumT_ref,
                bq_ref, bv_ref, bu_ref, out_ref, cur_ref):
    f32 = jnp.float32
    inv_scale = f32(1.0 / (_D ** 0.5))
    neg_inf = f32(-jnp.inf)

    for b in range(_B):
        cur_ref[b] = mem_ref[...]

    iota = jax.lax.broadcasted_iota(jnp.int32, (_B, _M), 1)

    def step(t, carry):
        xt = xs_ref[pl.ds(t, 1), :, :].reshape(_B, _D)
        q = jnp.dot(xt, wqT_ref[...], preferred_element_type=f32) + bq_ref[...]
        w = jnp.dot(q, wk_ref[...], preferred_element_type=f32)

        rows_l = [jax.lax.dot_general(w[b:b + 1, :], cur_ref[b],
                                      dimension_numbers=(((1,), (1,)), ((), ())),
                                      preferred_element_type=f32)
                  for b in range(_B)]
        logits = jnp.concatenate(rows_l, axis=0) * inv_scale

        mx = jnp.max(logits, axis=1, keepdims=True)
        e = jnp.exp(logits - mx)
        attn = e * (1.0 / jnp.sum(e, axis=1, keepdims=True))

        rows = [jnp.dot(attn[b:b + 1, :], cur_ref[b], preferred_element_type=f32)
                for b in range(_B)]
        s = jnp.concatenate(rows, axis=0)
        mem_out = jnp.dot(s, wvT_ref[...], preferred_element_type=f32) + bv_ref[...]
        out_ref[pl.ds(t, 1), :, :] = mem_out.reshape(1, _B, _D)

        u = jax.nn.sigmoid(jnp.dot(xt, wuxT_ref[...], preferred_element_type=f32)
                           + jnp.dot(mem_out, wumT_ref[...], preferred_element_type=f32)
                           + bu_ref[...])

        lg = logits
        top = []
        for _ in range(_TOPK):
            mxj = jnp.max(lg, axis=1, keepdims=True)
            cand = jnp.where(lg >= mxj, iota, _M)
            ij = jnp.min(cand, axis=1, keepdims=True)
            top.append(ij)
            lg = jnp.where(iota == ij, neg_inf, lg)

        for b in range(_B):
            ub = u[b:b + 1, :]
            xb = xt[b:b + 1, :]
            for j in range(_TOPK):
                i = top[j][b, 0]
                g = cur_ref[b, pl.ds(i, 1), :]
                cur_ref[b, pl.ds(i, 1), :] = g + ub * (xb - g)
        return carry

    jax.lax.fori_loop(0, _S, step, 0)


def _run(xs, memory, wqT, wk, wvT, wuxT, wumT, bq2, bv2, bu2):
    return pl.pallas_call(
        _layer_body,
        out_shape=jax.ShapeDtypeStruct((_S, _B, _D), jnp.float32),
        scratch_shapes=[pltpu.VMEM((_B, _M, _D), jnp.float32)],
    )(xs, memory, wqT, wk, wvT, wuxT, wumT, bq2, bv2, bu2)


def kernel(x, memory, Wq, bq, Wk, bk, Wv, bv, Wu, bu):
    del bk
    xs = jnp.transpose(x, (1, 0, 2))
    wqT = Wq.T
    wvT = Wv.T
    wuxT = Wu[:, :_D].T
    wumT = Wu[:, _D:].T
    bq2 = bq.reshape(1, _D)
    bv2 = bv.reshape(1, _D)
    bu2 = bu.reshape(1, _D)
    outs = _run(xs, memory, wqT, Wk, wvT, wuxT, wumT, bq2, bv2, bu2)
    return jnp.transpose(outs, (1, 0, 2))
